# baseline (speedup 1.0000x reference)
import functools

import jax
import jax.numpy as jnp
from jax import lax
from jax.experimental import pallas as pl
from jax.experimental.pallas import tpu as pltpu
from jax.experimental.pallas import tpu_sc as plsc

N = 10000
E = 320000
D_IN = 128
H = 64
C = 100
B = 16

NC = 2
NS = 16
NW = NC * NS
K = 128
NCHUNK = ((-(-E // (NW * K)) + 3) // 4) * 4
NCH0 = 68
NCH1 = 2 * NCHUNK - NCH0
NCHMAX = max(NCH0, NCH1)
TOTCH = NW * NCHUNK
EP = TOTCH * K
RPS = N // NS


def _mt(a, b):
    return lax.dot_general(a, b, (((1,), (1,)), ((), ())),
                           preferred_element_type=jnp.float32)


@functools.cache
def _mesh():
    return plsc.VectorSubcoreMesh(core_axis_name="c", subcore_axis_name="s",
                                  num_cores=NC, num_subcores=NS)


@functools.cache
def _deg_fn():
    @functools.partial(
        pl.kernel,
        out_type=jax.ShapeDtypeStruct((NW, N), jnp.float32),
        mesh=_mesh(),
        compiler_params=pltpu.CompilerParams(needs_layout_passes=False, use_tc_tiling_on_sc=False),
        scratch_types=[
            pltpu.VMEM((NCHUNK, K), jnp.int32),
            pltpu.VMEM((NCHUNK, K), jnp.float32),
            pltpu.VMEM((N,), jnp.float32),
        ],
    )
    def deg_kernel(colm_hbm, validm_hbm, out_hbm, colbuf, vbuf, cnt):
        c = lax.axis_index("c")
        s = lax.axis_index("s")
        w = c * NS + s
        z16 = jnp.zeros((16,), jnp.float32)

        def zb(i, carry):
            cnt[pl.ds(i * 16, 16)] = z16
            return carry
        lax.fori_loop(0, N // 16, zb, 0)

        pltpu.sync_copy(colm_hbm.at[pl.ds(w * NCHUNK, NCHUNK)], colbuf)
        pltpu.sync_copy(validm_hbm.at[pl.ds(w * NCHUNK, NCHUNK)], vbuf)

        def jb(j, carry):
            def ib(i, carry2):
                colv = colbuf[j, pl.ds(i * 16, 16)]
                vv = vbuf[j, pl.ds(i * 16, 16)]
                plsc.addupdate_scatter(cnt, [colv], vv)
                return carry2
            return lax.fori_loop(0, K // 16, ib, carry)
        lax.fori_loop(0, NCHUNK, jb, 0)

        pltpu.sync_copy(cnt, out_hbm.at[w])

    return deg_kernel


@functools.cache
def _agg_fn():
    @functools.partial(
        pl.kernel,
        out_type=jax.ShapeDtypeStruct((NC, N, H), jnp.float32),
        mesh=_mesh(),
        compiler_params=pltpu.CompilerParams(needs_layout_passes=False, use_tc_tiling_on_sc=False),
        scratch_types=[
            pltpu.VMEM((NCHMAX, K), jnp.int32),
            pltpu.VMEM((NCHMAX, K), jnp.int32),
            pltpu.VMEM((NCHMAX, K), jnp.float32),
            pltpu.VMEM((K, H), jnp.float32),
            pltpu.VMEM((K, H), jnp.float32),
            pltpu.VMEM((K, H), jnp.float32),
            pltpu.VMEM((K, H), jnp.float32),
            pltpu.VMEM_SHARED((N, H), jnp.float32),
            pltpu.SemaphoreType.DMA,
            pltpu.SemaphoreType.DMA,
            pltpu.SemaphoreType.DMA,
            pltpu.SemaphoreType.DMA,
            pltpu.SemaphoreType.DMA,
            pltpu.SemaphoreType.DMA,
            pltpu.SemaphoreType.DMA,
            pltpu.SemaphoreType.DMA,
        ],
    )
    def agg_kernel(hws_hbm, rowm_hbm, colm_hbm, wm_hbm, zeros_hbm, out_hbm,
                   rowbuf, colbuf, wbuf, g0, g1, g2, g3, acc,
                   gs0, gs1, gs2, gs3, ss0, ss1, ss2, ss3):
        c = lax.axis_index("c")
        s = lax.axis_index("s")
        w = c * NS + s
        gbufs = (g0, g1, g2, g3)
        gsems = (gs0, gs1, gs2, gs3)
        ssems = (ss0, ss1, ss2, ss3)

        pltpu.sync_copy(zeros_hbm.at[pl.ds(s * RPS, RPS)],
                        acc.at[pl.ds(s * RPS, RPS)])

        def issue_gather(slot, j):
            pltpu.async_copy(hws_hbm.at[rowbuf.at[j]], gbufs[slot],
                             gsems[slot])

        def wait_gather(slot, j):
            pltpu.make_async_copy(hws_hbm.at[rowbuf.at[j]], gbufs[slot],
                                  gsems[slot]).wait()

        def issue_scatter(slot, j):
            pltpu.async_copy(gbufs[slot], acc.at[colbuf.at[j]], ssems[slot],
                             add=True)

        def wait_scatter(slot, j):
            pltpu.make_async_copy(gbufs[slot], acc.at[colbuf.at[j]],
                                  ssems[slot]).wait()

        def scale(slot, j):
            gbuf = gbufs[slot]

            @plsc.parallel_loop(0, K // 16, unroll=4)
            def _(g):
                wv = wbuf[j, pl.ds(g * 16, 16)]
                for l in range(16):
                    sc = wv[l]
                    base = g * 16 + l
                    for q in range(H // 16):
                        gv = gbuf[base, pl.ds(q * 16, 16)]
                        gbuf[base, pl.ds(q * 16, 16)] = gv * sc

        def run_pipeline(nch, start):
            pltpu.sync_copy(rowm_hbm.at[pl.ds(start, nch)],
                            rowbuf.at[pl.ds(0, nch)])
            pltpu.sync_copy(colm_hbm.at[pl.ds(start, nch)],
                            colbuf.at[pl.ds(0, nch)])
            pltpu.sync_copy(wm_hbm.at[pl.ds(start, nch)],
                            wbuf.at[pl.ds(0, nch)])
            issue_gather(0, 0)
            issue_gather(1, 1)

            def body(i, carry):
                for b in range(4):
                    j = 4 * i + b
                    s2 = (b + 2) % 4
                    if b < 2:
                        @pl.when(i > 0)
                        def _():
                            wait_scatter(s2, j - 2)
                        issue_gather(s2, j + 2)
                    else:
                        wait_scatter(s2, j - 2)

                        @pl.when(i < nch // 4 - 1)
                        def _():
                            issue_gather(s2, j + 2)
                    wait_gather(b, j)
                    scale(b, j)
                    issue_scatter(b, j)
                return carry
            lax.fori_loop(0, nch // 4, body, 0)
            wait_scatter(2, nch - 2)
            wait_scatter(3, nch - 1)

        pair_base = s * (NCH0 + NCH1)

        @pl.when(c == 0)
        def _():
            run_pipeline(NCH0, pair_base)

        @pl.when(c == 1)
        def _():
            run_pipeline(NCH1, pair_base + NCH0)

        plsc.subcore_barrier()
        pltpu.sync_copy(acc.at[pl.ds(s * RPS, RPS)],
                        out_hbm.at[c, pl.ds(s * RPS, RPS)])

    return agg_kernel


def _prep_body(degparts_ref, eap_ref, valid_ref, dinv_ref, wedge_ref):
    deg = jnp.sum(degparts_ref[...], axis=0, keepdims=True) + 1.0
    dinv_ref[...] = lax.rsqrt(deg)
    wedge_ref[...] = jnp.exp(-eap_ref[...]) * valid_ref[...]


def _mm1_body(x_ref, w1_ref, dinvc_ref, out_ref):
    out_ref[...] = _mt(x_ref[...], w1_ref[...]) * dinvc_ref[...]


def _layer_body(p_ref, hws_ref, dinvc_ref, w_ref, b_ref, out_ref):
    t = p_ref[0] + p_ref[1] + hws_ref[...]
    h = jnp.maximum(dinvc_ref[...] * t + b_ref[...], 0.0)
    out_ref[...] = _mt(h, w_ref[...]) * dinvc_ref[...]


def _final_body(p_ref, hws_ref, dinvc_ref, b_ref, batch_ref, wc_ref, bc_ref,
                out_ref):
    t = p_ref[0] + p_ref[1] + hws_ref[...]
    h = jnp.maximum(dinvc_ref[...] * t + b_ref[...], 0.0)
    ids = lax.broadcasted_iota(jnp.int32, (B, N), 0)
    onehot = (batch_ref[...] == ids).astype(jnp.float32)
    sums = lax.dot_general(onehot, h, (((1,), (0,)), ((), ())),
                           preferred_element_type=jnp.float32)
    cnt = jnp.sum(onehot, axis=1, keepdims=True)
    pooled = sums / jnp.maximum(cnt, 1.0)
    out_ref[...] = _mt(pooled, wc_ref[...]) + bc_ref[...]


@jax.jit
def kernel(x, edge_index, edge_attr, batch, W1, b1, W2, b2, W3, b3, Wc, bc):
    row = edge_index[0]
    col = edge_index[1]
    pad = EP - E
    rowm = jnp.concatenate([row, jnp.zeros((pad,), row.dtype)]
                           ).reshape(TOTCH, K)
    colm = jnp.concatenate([col, jnp.zeros((pad,), col.dtype)]
                           ).reshape(TOTCH, K)
    eap = jnp.concatenate([edge_attr, jnp.zeros((pad,), edge_attr.dtype)]
                          ).reshape(EP // K, K)
    valid = jnp.concatenate([jnp.ones((E,), jnp.float32),
                             jnp.zeros((pad,), jnp.float32)])
    validm = valid.reshape(TOTCH, K)
    valid2 = validm
    zeros_nh = jnp.zeros((N, H), jnp.float32)

    degparts = _deg_fn()(colm, validm)

    dinv_row, wedge2 = pl.pallas_call(
        _prep_body,
        out_shape=[jax.ShapeDtypeStruct((1, N), jnp.float32),
                   jax.ShapeDtypeStruct((EP // K, K), jnp.float32)],
    )(degparts, eap, valid2)
    dinvc = dinv_row.reshape(N, 1)
    wm = wedge2

    hws = pl.pallas_call(
        _mm1_body,
        out_shape=jax.ShapeDtypeStruct((N, H), jnp.float32),
    )(x, W1, dinvc)

    for (Wn, bn) in ((W2, b1), (W3, b2)):
        parts = _agg_fn()(hws, rowm, colm, wm, zeros_nh)
        hws = pl.pallas_call(
            _layer_body,
            out_shape=jax.ShapeDtypeStruct((N, H), jnp.float32),
        )(parts, hws, dinvc, Wn, bn.reshape(1, H))

    parts = _agg_fn()(hws, rowm, colm, wm, zeros_nh)
    out = pl.pallas_call(
        _final_body,
        out_shape=jax.ShapeDtypeStruct((B, C), jnp.float32),
    )(parts, hws, dinvc, b3.reshape(1, H), batch.reshape(1, N), Wc,
      bc.reshape(1, C))
    return out


# speedup vs baseline: 1.0385x; 1.0385x over previous
import functools

import jax
import jax.numpy as jnp
from jax import lax
from jax.experimental import pallas as pl
from jax.experimental.pallas import tpu as pltpu
from jax.experimental.pallas import tpu_sc as plsc

N = 10000
E = 320000
D_IN = 128
H = 64
C = 100
B = 16

NC = 2
NS = 16
NW = NC * NS
K = 128
NCHUNK = ((-(-E // (NW * K)) + 3) // 4) * 4
NCH0 = 92
NCH1 = 2 * NCHUNK - NCH0
NCHMAX = max(NCH0, NCH1)
TOTCH = NW * NCHUNK
EP = TOTCH * K
RPS = N // NS


def _mt(a, b):
    return lax.dot_general(a, b, (((1,), (1,)), ((), ())),
                           preferred_element_type=jnp.float32)


@functools.cache
def _mesh():
    return plsc.VectorSubcoreMesh(core_axis_name="c", subcore_axis_name="s",
                                  num_cores=NC, num_subcores=NS)


@functools.cache
def _deg_fn():
    @functools.partial(
        pl.kernel,
        out_type=jax.ShapeDtypeStruct((NW, N), jnp.float32),
        mesh=_mesh(),
        compiler_params=pltpu.CompilerParams(needs_layout_passes=False, use_tc_tiling_on_sc=False),
        scratch_types=[
            pltpu.VMEM((NCHUNK, K), jnp.int32),
            pltpu.VMEM((NCHUNK, K), jnp.float32),
            pltpu.VMEM((N,), jnp.float32),
        ],
    )
    def deg_kernel(colm_hbm, validm_hbm, out_hbm, colbuf, vbuf, cnt):
        c = lax.axis_index("c")
        s = lax.axis_index("s")
        w = c * NS + s
        z16 = jnp.zeros((16,), jnp.float32)

        def zb(i, carry):
            cnt[pl.ds(i * 16, 16)] = z16
            return carry
        lax.fori_loop(0, N // 16, zb, 0)

        pltpu.sync_copy(colm_hbm.at[pl.ds(w * NCHUNK, NCHUNK)], colbuf)
        pltpu.sync_copy(validm_hbm.at[pl.ds(w * NCHUNK, NCHUNK)], vbuf)

        def jb(j, carry):
            def ib(i, carry2):
                colv = colbuf[j, pl.ds(i * 16, 16)]
                vv = vbuf[j, pl.ds(i * 16, 16)]
                plsc.addupdate_scatter(cnt, [colv], vv)
                return carry2
            return lax.fori_loop(0, K // 16, ib, carry)
        lax.fori_loop(0, NCHUNK, jb, 0)

        pltpu.sync_copy(cnt, out_hbm.at[w])

    return deg_kernel


@functools.cache
def _agg_fn():
    @functools.partial(
        pl.kernel,
        out_type=jax.ShapeDtypeStruct((NC, N, H), jnp.float32),
        mesh=_mesh(),
        compiler_params=pltpu.CompilerParams(needs_layout_passes=False, use_tc_tiling_on_sc=False),
        scratch_types=[
            pltpu.VMEM((NCHMAX, K), jnp.int32),
            pltpu.VMEM((NCHMAX, K), jnp.int32),
            pltpu.VMEM((NCHMAX, K), jnp.float32),
            pltpu.VMEM((K, H), jnp.float32),
            pltpu.VMEM((K, H), jnp.float32),
            pltpu.VMEM((K, H), jnp.float32),
            pltpu.VMEM((K, H), jnp.float32),
            pltpu.VMEM_SHARED((N, H), jnp.float32),
            pltpu.SemaphoreType.DMA,
            pltpu.SemaphoreType.DMA,
            pltpu.SemaphoreType.DMA,
            pltpu.SemaphoreType.DMA,
            pltpu.SemaphoreType.DMA,
            pltpu.SemaphoreType.DMA,
            pltpu.SemaphoreType.DMA,
            pltpu.SemaphoreType.DMA,
        ],
    )
    def agg_kernel(hws_hbm, rowm_hbm, colm_hbm, wm_hbm, zeros_hbm, out_hbm,
                   rowbuf, colbuf, wbuf, g0, g1, g2, g3, acc,
                   gs0, gs1, gs2, gs3, ss0, ss1, ss2, ss3):
        c = lax.axis_index("c")
        s = lax.axis_index("s")
        w = c * NS + s
        gbufs = (g0, g1, g2, g3)
        gsems = (gs0, gs1, gs2, gs3)
        ssems = (ss0, ss1, ss2, ss3)

        pltpu.sync_copy(zeros_hbm.at[pl.ds(s * RPS, RPS)],
                        acc.at[pl.ds(s * RPS, RPS)])

        def issue_gather(slot, j):
            pltpu.async_copy(hws_hbm.at[rowbuf.at[j]], gbufs[slot],
                             gsems[slot])

        def wait_gather(slot, j):
            pltpu.make_async_copy(hws_hbm.at[rowbuf.at[j]], gbufs[slot],
                                  gsems[slot]).wait()

        def issue_scatter(slot, j):
            pltpu.async_copy(gbufs[slot], acc.at[colbuf.at[j]], ssems[slot],
                             add=True)

        def wait_scatter(slot, j):
            pltpu.make_async_copy(gbufs[slot], acc.at[colbuf.at[j]],
                                  ssems[slot]).wait()

        def scale(slot, j):
            gbuf = gbufs[slot]

            @plsc.parallel_loop(0, K // 16, unroll=4)
            def _(g):
                wv = wbuf[j, pl.ds(g * 16, 16)]
                for l in range(16):
                    sc = wv[l]
                    base = g * 16 + l
                    for q in range(H // 16):
                        gv = gbuf[base, pl.ds(q * 16, 16)]
                        gbuf[base, pl.ds(q * 16, 16)] = gv * sc

        def run_pipeline(nch, start):
            pltpu.sync_copy(rowm_hbm.at[pl.ds(start, nch)],
                            rowbuf.at[pl.ds(0, nch)])
            pltpu.sync_copy(colm_hbm.at[pl.ds(start, nch)],
                            colbuf.at[pl.ds(0, nch)])
            pltpu.sync_copy(wm_hbm.at[pl.ds(start, nch)],
                            wbuf.at[pl.ds(0, nch)])
            issue_gather(0, 0)
            issue_gather(1, 1)

            def body(i, carry):
                for b in range(4):
                    j = 4 * i + b
                    s2 = (b + 2) % 4
                    if b < 2:
                        @pl.when(i > 0)
                        def _():
                            wait_scatter(s2, j - 2)
                        issue_gather(s2, j + 2)
                    else:
                        wait_scatter(s2, j - 2)

                        @pl.when(i < nch // 4 - 1)
                        def _():
                            issue_gather(s2, j + 2)
                    wait_gather(b, j)
                    scale(b, j)
                    issue_scatter(b, j)
                return carry
            lax.fori_loop(0, nch // 4, body, 0)
            wait_scatter(2, nch - 2)
            wait_scatter(3, nch - 1)

        pair_base = s * (NCH0 + NCH1)

        @pl.when(c == 0)
        def _():
            run_pipeline(NCH0, pair_base)

        @pl.when(c == 1)
        def _():
            run_pipeline(NCH1, pair_base + NCH0)

        plsc.subcore_barrier()
        pltpu.sync_copy(acc.at[pl.ds(s * RPS, RPS)],
                        out_hbm.at[c, pl.ds(s * RPS, RPS)])

    return agg_kernel


def _prep_body(degparts_ref, eap_ref, valid_ref, dinv_ref, wedge_ref):
    deg = jnp.sum(degparts_ref[...], axis=0, keepdims=True) + 1.0
    dinv_ref[...] = lax.rsqrt(deg)
    wedge_ref[...] = jnp.exp(-eap_ref[...]) * valid_ref[...]


def _mm1_body(x_ref, w1_ref, dinvc_ref, out_ref):
    out_ref[...] = _mt(x_ref[...], w1_ref[...]) * dinvc_ref[...]


def _layer_body(p_ref, hws_ref, dinvc_ref, w_ref, b_ref, out_ref):
    t = p_ref[0] + p_ref[1] + hws_ref[...]
    h = jnp.maximum(dinvc_ref[...] * t + b_ref[...], 0.0)
    out_ref[...] = _mt(h, w_ref[...]) * dinvc_ref[...]


def _final_body(p_ref, hws_ref, dinvc_ref, b_ref, batch_ref, wc_ref, bc_ref,
                out_ref):
    t = p_ref[0] + p_ref[1] + hws_ref[...]
    h = jnp.maximum(dinvc_ref[...] * t + b_ref[...], 0.0)
    ids = lax.broadcasted_iota(jnp.int32, (B, N), 0)
    onehot = (batch_ref[...] == ids).astype(jnp.float32)
    sums = lax.dot_general(onehot, h, (((1,), (0,)), ((), ())),
                           preferred_element_type=jnp.float32)
    cnt = jnp.sum(onehot, axis=1, keepdims=True)
    pooled = sums / jnp.maximum(cnt, 1.0)
    out_ref[...] = _mt(pooled, wc_ref[...]) + bc_ref[...]


@jax.jit
def kernel(x, edge_index, edge_attr, batch, W1, b1, W2, b2, W3, b3, Wc, bc):
    row = edge_index[0]
    col = edge_index[1]
    pad = EP - E
    rowm = jnp.concatenate([row, jnp.zeros((pad,), row.dtype)]
                           ).reshape(TOTCH, K)
    colm = jnp.concatenate([col, jnp.zeros((pad,), col.dtype)]
                           ).reshape(TOTCH, K)
    eap = jnp.concatenate([edge_attr, jnp.zeros((pad,), edge_attr.dtype)]
                          ).reshape(EP // K, K)
    valid = jnp.concatenate([jnp.ones((E,), jnp.float32),
                             jnp.zeros((pad,), jnp.float32)])
    validm = valid.reshape(TOTCH, K)
    valid2 = validm
    zeros_nh = jnp.zeros((N, H), jnp.float32)

    degparts = _deg_fn()(colm, validm)

    dinv_row, wedge2 = pl.pallas_call(
        _prep_body,
        out_shape=[jax.ShapeDtypeStruct((1, N), jnp.float32),
                   jax.ShapeDtypeStruct((EP // K, K), jnp.float32)],
    )(degparts, eap, valid2)
    dinvc = dinv_row.reshape(N, 1)
    wm = wedge2

    hws = pl.pallas_call(
        _mm1_body,
        out_shape=jax.ShapeDtypeStruct((N, H), jnp.float32),
    )(x, W1, dinvc)

    for (Wn, bn) in ((W2, b1), (W3, b2)):
        parts = _agg_fn()(hws, rowm, colm, wm, zeros_nh)
        hws = pl.pallas_call(
            _layer_body,
            out_shape=jax.ShapeDtypeStruct((N, H), jnp.float32),
        )(parts, hws, dinvc, Wn, bn.reshape(1, H))

    parts = _agg_fn()(hws, rowm, colm, wm, zeros_nh)
    out = pl.pallas_call(
        _final_body,
        out_shape=jax.ShapeDtypeStruct((B, C), jnp.float32),
    )(parts, hws, dinvc, b3.reshape(1, H), batch.reshape(1, N), Wc,
      bc.reshape(1, C))
    return out


# speedup vs baseline: 1.0467x; 1.0079x over previous
import functools

import jax
import jax.numpy as jnp
from jax import lax
from jax.experimental import pallas as pl
from jax.experimental.pallas import tpu as pltpu
from jax.experimental.pallas import tpu_sc as plsc

N = 10000
E = 320000
D_IN = 128
H = 64
C = 100
B = 16

NC = 2
NS = 16
NW = NC * NS
K = 128
NCHUNK = ((-(-E // (NW * K)) + 3) // 4) * 4
NCH0 = 96
NCH1 = 2 * NCHUNK - NCH0
NCHMAX = max(NCH0, NCH1)
TOTCH = NW * NCHUNK
EP = TOTCH * K
RPS = N // NS


def _mt(a, b):
    return lax.dot_general(a, b, (((1,), (1,)), ((), ())),
                           preferred_element_type=jnp.float32)


@functools.cache
def _mesh():
    return plsc.VectorSubcoreMesh(core_axis_name="c", subcore_axis_name="s",
                                  num_cores=NC, num_subcores=NS)


@functools.cache
def _deg_fn():
    @functools.partial(
        pl.kernel,
        out_type=jax.ShapeDtypeStruct((NW, N), jnp.float32),
        mesh=_mesh(),
        compiler_params=pltpu.CompilerParams(needs_layout_passes=False, use_tc_tiling_on_sc=False),
        scratch_types=[
            pltpu.VMEM((NCHUNK, K), jnp.int32),
            pltpu.VMEM((NCHUNK, K), jnp.float32),
            pltpu.VMEM((N,), jnp.float32),
        ],
    )
    def deg_kernel(colm_hbm, validm_hbm, out_hbm, colbuf, vbuf, cnt):
        c = lax.axis_index("c")
        s = lax.axis_index("s")
        w = c * NS + s
        z16 = jnp.zeros((16,), jnp.float32)

        def zb(i, carry):
            cnt[pl.ds(i * 16, 16)] = z16
            return carry
        lax.fori_loop(0, N // 16, zb, 0)

        pltpu.sync_copy(colm_hbm.at[pl.ds(w * NCHUNK, NCHUNK)], colbuf)
        pltpu.sync_copy(validm_hbm.at[pl.ds(w * NCHUNK, NCHUNK)], vbuf)

        def jb(j, carry):
            def ib(i, carry2):
                colv = colbuf[j, pl.ds(i * 16, 16)]
                vv = vbuf[j, pl.ds(i * 16, 16)]
                plsc.addupdate_scatter(cnt, [colv], vv)
                return carry2
            return lax.fori_loop(0, K // 16, ib, carry)
        lax.fori_loop(0, NCHUNK, jb, 0)

        pltpu.sync_copy(cnt, out_hbm.at[w])

    return deg_kernel


@functools.cache
def _agg_fn():
    @functools.partial(
        pl.kernel,
        out_type=jax.ShapeDtypeStruct((NC, N, H), jnp.float32),
        mesh=_mesh(),
        compiler_params=pltpu.CompilerParams(needs_layout_passes=False, use_tc_tiling_on_sc=False),
        scratch_types=[
            pltpu.VMEM((NCHMAX, K), jnp.int32),
            pltpu.VMEM((NCHMAX, K), jnp.int32),
            pltpu.VMEM((NCHMAX, K), jnp.float32),
            pltpu.VMEM((K, H), jnp.float32),
            pltpu.VMEM((K, H), jnp.float32),
            pltpu.VMEM((K, H), jnp.float32),
            pltpu.VMEM((K, H), jnp.float32),
            pltpu.VMEM_SHARED((N, H), jnp.float32),
            pltpu.SemaphoreType.DMA,
            pltpu.SemaphoreType.DMA,
            pltpu.SemaphoreType.DMA,
            pltpu.SemaphoreType.DMA,
            pltpu.SemaphoreType.DMA,
            pltpu.SemaphoreType.DMA,
            pltpu.SemaphoreType.DMA,
            pltpu.SemaphoreType.DMA,
        ],
    )
    def agg_kernel(hws_hbm, rowm_hbm, colm_hbm, wm_hbm, zeros_hbm, out_hbm,
                   rowbuf, colbuf, wbuf, g0, g1, g2, g3, acc,
                   gs0, gs1, gs2, gs3, ss0, ss1, ss2, ss3):
        c = lax.axis_index("c")
        s = lax.axis_index("s")
        w = c * NS + s
        gbufs = (g0, g1, g2, g3)
        gsems = (gs0, gs1, gs2, gs3)
        ssems = (ss0, ss1, ss2, ss3)

        pltpu.sync_copy(zeros_hbm.at[pl.ds(s * RPS, RPS)],
                        acc.at[pl.ds(s * RPS, RPS)])

        def issue_gather(slot, j):
            pltpu.async_copy(hws_hbm.at[rowbuf.at[j]], gbufs[slot],
                             gsems[slot])

        def wait_gather(slot, j):
            pltpu.make_async_copy(hws_hbm.at[rowbuf.at[j]], gbufs[slot],
                                  gsems[slot]).wait()

        def issue_scatter(slot, j):
            pltpu.async_copy(gbufs[slot], acc.at[colbuf.at[j]], ssems[slot],
                             add=True)

        def wait_scatter(slot, j):
            pltpu.make_async_copy(gbufs[slot], acc.at[colbuf.at[j]],
                                  ssems[slot]).wait()

        def scale(slot, j):
            gbuf = gbufs[slot]

            @plsc.parallel_loop(0, K // 16, unroll=4)
            def _(g):
                wv = wbuf[j, pl.ds(g * 16, 16)]
                for l in range(16):
                    sc = wv[l]
                    base = g * 16 + l
                    for q in range(H // 16):
                        gv = gbuf[base, pl.ds(q * 16, 16)]
                        gbuf[base, pl.ds(q * 16, 16)] = gv * sc

        def run_pipeline(nch, start):
            pltpu.sync_copy(rowm_hbm.at[pl.ds(start, nch)],
                            rowbuf.at[pl.ds(0, nch)])
            pltpu.sync_copy(colm_hbm.at[pl.ds(start, nch)],
                            colbuf.at[pl.ds(0, nch)])
            pltpu.sync_copy(wm_hbm.at[pl.ds(start, nch)],
                            wbuf.at[pl.ds(0, nch)])
            issue_gather(0, 0)
            issue_gather(1, 1)

            def body(i, carry):
                for b in range(4):
                    j = 4 * i + b
                    s2 = (b + 2) % 4
                    if b < 2:
                        @pl.when(i > 0)
                        def _():
                            wait_scatter(s2, j - 2)
                        issue_gather(s2, j + 2)
                    else:
                        wait_scatter(s2, j - 2)

                        @pl.when(i < nch // 4 - 1)
                        def _():
                            issue_gather(s2, j + 2)
                    wait_gather(b, j)
                    scale(b, j)
                    issue_scatter(b, j)
                return carry
            lax.fori_loop(0, nch // 4, body, 0)
            wait_scatter(2, nch - 2)
            wait_scatter(3, nch - 1)

        pair_base = s * (NCH0 + NCH1)

        @pl.when(c == 0)
        def _():
            run_pipeline(NCH0, pair_base)

        @pl.when(c == 1)
        def _():
            run_pipeline(NCH1, pair_base + NCH0)

        plsc.subcore_barrier()
        pltpu.sync_copy(acc.at[pl.ds(s * RPS, RPS)],
                        out_hbm.at[c, pl.ds(s * RPS, RPS)])

    return agg_kernel


def _prep_body(degparts_ref, eap_ref, valid_ref, dinv_ref, wedge_ref):
    deg = jnp.sum(degparts_ref[...], axis=0, keepdims=True) + 1.0
    dinv_ref[...] = lax.rsqrt(deg)
    wedge_ref[...] = jnp.exp(-eap_ref[...]) * valid_ref[...]


def _mm1_body(x_ref, w1_ref, dinvc_ref, out_ref):
    out_ref[...] = _mt(x_ref[...], w1_ref[...]) * dinvc_ref[...]


def _layer_body(p_ref, hws_ref, dinvc_ref, w_ref, b_ref, out_ref):
    t = p_ref[0] + p_ref[1] + hws_ref[...]
    h = jnp.maximum(dinvc_ref[...] * t + b_ref[...], 0.0)
    out_ref[...] = _mt(h, w_ref[...]) * dinvc_ref[...]


def _final_body(p_ref, hws_ref, dinvc_ref, b_ref, batch_ref, wc_ref, bc_ref,
                out_ref):
    t = p_ref[0] + p_ref[1] + hws_ref[...]
    h = jnp.maximum(dinvc_ref[...] * t + b_ref[...], 0.0)
    ids = lax.broadcasted_iota(jnp.int32, (B, N), 0)
    onehot = (batch_ref[...] == ids).astype(jnp.float32)
    sums = lax.dot_general(onehot, h, (((1,), (0,)), ((), ())),
                           preferred_element_type=jnp.float32)
    cnt = jnp.sum(onehot, axis=1, keepdims=True)
    pooled = sums / jnp.maximum(cnt, 1.0)
    out_ref[...] = _mt(pooled, wc_ref[...]) + bc_ref[...]


@jax.jit
def kernel(x, edge_index, edge_attr, batch, W1, b1, W2, b2, W3, b3, Wc, bc):
    row = edge_index[0]
    col = edge_index[1]
    pad = EP - E
    rowm = jnp.concatenate([row, jnp.zeros((pad,), row.dtype)]
                           ).reshape(TOTCH, K)
    colm = jnp.concatenate([col, jnp.zeros((pad,), col.dtype)]
                           ).reshape(TOTCH, K)
    eap = jnp.concatenate([edge_attr, jnp.zeros((pad,), edge_attr.dtype)]
                          ).reshape(EP // K, K)
    valid = jnp.concatenate([jnp.ones((E,), jnp.float32),
                             jnp.zeros((pad,), jnp.float32)])
    validm = valid.reshape(TOTCH, K)
    valid2 = validm
    zeros_nh = jnp.zeros((N, H), jnp.float32)

    degparts = _deg_fn()(colm, validm)

    dinv_row, wedge2 = pl.pallas_call(
        _prep_body,
        out_shape=[jax.ShapeDtypeStruct((1, N), jnp.float32),
                   jax.ShapeDtypeStruct((EP // K, K), jnp.float32)],
    )(degparts, eap, valid2)
    dinvc = dinv_row.reshape(N, 1)
    wm = wedge2

    hws = pl.pallas_call(
        _mm1_body,
        out_shape=jax.ShapeDtypeStruct((N, H), jnp.float32),
    )(x, W1, dinvc)

    for (Wn, bn) in ((W2, b1), (W3, b2)):
        parts = _agg_fn()(hws, rowm, colm, wm, zeros_nh)
        hws = pl.pallas_call(
            _layer_body,
            out_shape=jax.ShapeDtypeStruct((N, H), jnp.float32),
        )(parts, hws, dinvc, Wn, bn.reshape(1, H))

    parts = _agg_fn()(hws, rowm, colm, wm, zeros_nh)
    out = pl.pallas_call(
        _final_body,
        out_shape=jax.ShapeDtypeStruct((B, C), jnp.float32),
    )(parts, hws, dinvc, b3.reshape(1, H), batch.reshape(1, N), Wc,
      bc.reshape(1, C))
    return out


# speedup vs baseline: 1.0870x; 1.0385x over previous
import functools

import jax
import jax.numpy as jnp
from jax import lax
from jax.experimental import pallas as pl
from jax.experimental.pallas import tpu as pltpu
from jax.experimental.pallas import tpu_sc as plsc

N = 10000
E = 320000
D_IN = 128
H = 64
C = 100
B = 16

NC = 2
NS = 16
NW = NC * NS
K = 128
NCHUNK = ((-(-E // (NW * K)) + 3) // 4) * 4
NCH0 = 104
NCH1 = 2 * NCHUNK - NCH0
NCHMAX = max(NCH0, NCH1)
TOTCH = NW * NCHUNK
EP = TOTCH * K
RPS = N // NS


def _mt(a, b):
    return lax.dot_general(a, b, (((1,), (1,)), ((), ())),
                           preferred_element_type=jnp.float32)


@functools.cache
def _mesh():
    return plsc.VectorSubcoreMesh(core_axis_name="c", subcore_axis_name="s",
                                  num_cores=NC, num_subcores=NS)


@functools.cache
def _deg_fn():
    @functools.partial(
        pl.kernel,
        out_type=jax.ShapeDtypeStruct((NW, N), jnp.float32),
        mesh=_mesh(),
        compiler_params=pltpu.CompilerParams(needs_layout_passes=False, use_tc_tiling_on_sc=False),
        scratch_types=[
            pltpu.VMEM((NCHUNK, K), jnp.int32),
            pltpu.VMEM((NCHUNK, K), jnp.float32),
            pltpu.VMEM((N,), jnp.float32),
        ],
    )
    def deg_kernel(colm_hbm, validm_hbm, out_hbm, colbuf, vbuf, cnt):
        c = lax.axis_index("c")
        s = lax.axis_index("s")
        w = c * NS + s
        z16 = jnp.zeros((16,), jnp.float32)

        def zb(i, carry):
            cnt[pl.ds(i * 16, 16)] = z16
            return carry
        lax.fori_loop(0, N // 16, zb, 0)

        pltpu.sync_copy(colm_hbm.at[pl.ds(w * NCHUNK, NCHUNK)], colbuf)
        pltpu.sync_copy(validm_hbm.at[pl.ds(w * NCHUNK, NCHUNK)], vbuf)

        def jb(j, carry):
            def ib(i, carry2):
                colv = colbuf[j, pl.ds(i * 16, 16)]
                vv = vbuf[j, pl.ds(i * 16, 16)]
                plsc.addupdate_scatter(cnt, [colv], vv)
                return carry2
            return lax.fori_loop(0, K // 16, ib, carry)
        lax.fori_loop(0, NCHUNK, jb, 0)

        pltpu.sync_copy(cnt, out_hbm.at[w])

    return deg_kernel


@functools.cache
def _agg_fn():
    @functools.partial(
        pl.kernel,
        out_type=jax.ShapeDtypeStruct((NC, N, H), jnp.float32),
        mesh=_mesh(),
        compiler_params=pltpu.CompilerParams(needs_layout_passes=False, use_tc_tiling_on_sc=False),
        scratch_types=[
            pltpu.VMEM((NCHMAX, K), jnp.int32),
            pltpu.VMEM((NCHMAX, K), jnp.int32),
            pltpu.VMEM((NCHMAX, K), jnp.float32),
            pltpu.VMEM((K, H), jnp.float32),
            pltpu.VMEM((K, H), jnp.float32),
            pltpu.VMEM((K, H), jnp.float32),
            pltpu.VMEM((K, H), jnp.float32),
            pltpu.VMEM_SHARED((N, H), jnp.float32),
            pltpu.SemaphoreType.DMA,
            pltpu.SemaphoreType.DMA,
            pltpu.SemaphoreType.DMA,
            pltpu.SemaphoreType.DMA,
            pltpu.SemaphoreType.DMA,
            pltpu.SemaphoreType.DMA,
            pltpu.SemaphoreType.DMA,
            pltpu.SemaphoreType.DMA,
        ],
    )
    def agg_kernel(hws_hbm, rowm_hbm, colm_hbm, wm_hbm, zeros_hbm, out_hbm,
                   rowbuf, colbuf, wbuf, g0, g1, g2, g3, acc,
                   gs0, gs1, gs2, gs3, ss0, ss1, ss2, ss3):
        c = lax.axis_index("c")
        s = lax.axis_index("s")
        w = c * NS + s
        gbufs = (g0, g1, g2, g3)
        gsems = (gs0, gs1, gs2, gs3)
        ssems = (ss0, ss1, ss2, ss3)

        pltpu.sync_copy(zeros_hbm.at[pl.ds(s * RPS, RPS)],
                        acc.at[pl.ds(s * RPS, RPS)])

        def issue_gather(slot, j):
            pltpu.async_copy(hws_hbm.at[rowbuf.at[j]], gbufs[slot],
                             gsems[slot])

        def wait_gather(slot, j):
            pltpu.make_async_copy(hws_hbm.at[rowbuf.at[j]], gbufs[slot],
                                  gsems[slot]).wait()

        def issue_scatter(slot, j):
            pltpu.async_copy(gbufs[slot], acc.at[colbuf.at[j]], ssems[slot],
                             add=True)

        def wait_scatter(slot, j):
            pltpu.make_async_copy(gbufs[slot], acc.at[colbuf.at[j]],
                                  ssems[slot]).wait()

        def scale(slot, j):
            gbuf = gbufs[slot]

            @plsc.parallel_loop(0, K // 16, unroll=4)
            def _(g):
                wv = wbuf[j, pl.ds(g * 16, 16)]
                for l in range(16):
                    sc = wv[l]
                    base = g * 16 + l
                    for q in range(H // 16):
                        gv = gbuf[base, pl.ds(q * 16, 16)]
                        gbuf[base, pl.ds(q * 16, 16)] = gv * sc

        def run_pipeline(nch, start):
            pltpu.sync_copy(rowm_hbm.at[pl.ds(start, nch)],
                            rowbuf.at[pl.ds(0, nch)])
            pltpu.sync_copy(colm_hbm.at[pl.ds(start, nch)],
                            colbuf.at[pl.ds(0, nch)])
            pltpu.sync_copy(wm_hbm.at[pl.ds(start, nch)],
                            wbuf.at[pl.ds(0, nch)])
            issue_gather(0, 0)
            issue_gather(1, 1)

            def body(i, carry):
                for b in range(4):
                    j = 4 * i + b
                    s2 = (b + 2) % 4
                    if b < 2:
                        @pl.when(i > 0)
                        def _():
                            wait_scatter(s2, j - 2)
                        issue_gather(s2, j + 2)
                    else:
                        wait_scatter(s2, j - 2)

                        @pl.when(i < nch // 4 - 1)
                        def _():
                            issue_gather(s2, j + 2)
                    wait_gather(b, j)
                    scale(b, j)
                    issue_scatter(b, j)
                return carry
            lax.fori_loop(0, nch // 4, body, 0)
            wait_scatter(2, nch - 2)
            wait_scatter(3, nch - 1)

        pair_base = s * (NCH0 + NCH1)

        @pl.when(c == 0)
        def _():
            run_pipeline(NCH0, pair_base)

        @pl.when(c == 1)
        def _():
            run_pipeline(NCH1, pair_base + NCH0)

        plsc.subcore_barrier()
        pltpu.sync_copy(acc.at[pl.ds(s * RPS, RPS)],
                        out_hbm.at[c, pl.ds(s * RPS, RPS)])

    return agg_kernel


def _prep_body(degparts_ref, eap_ref, valid_ref, dinv_ref, wedge_ref):
    deg = jnp.sum(degparts_ref[...], axis=0, keepdims=True) + 1.0
    dinv_ref[...] = lax.rsqrt(deg)
    wedge_ref[...] = jnp.exp(-eap_ref[...]) * valid_ref[...]


def _mm1_body(x_ref, w1_ref, dinvc_ref, out_ref):
    out_ref[...] = _mt(x_ref[...], w1_ref[...]) * dinvc_ref[...]


def _layer_body(p_ref, hws_ref, dinvc_ref, w_ref, b_ref, out_ref):
    t = p_ref[0] + p_ref[1] + hws_ref[...]
    h = jnp.maximum(dinvc_ref[...] * t + b_ref[...], 0.0)
    out_ref[...] = _mt(h, w_ref[...]) * dinvc_ref[...]


def _final_body(p_ref, hws_ref, dinvc_ref, b_ref, batch_ref, wc_ref, bc_ref,
                out_ref):
    t = p_ref[0] + p_ref[1] + hws_ref[...]
    h = jnp.maximum(dinvc_ref[...] * t + b_ref[...], 0.0)
    ids = lax.broadcasted_iota(jnp.int32, (B, N), 0)
    onehot = (batch_ref[...] == ids).astype(jnp.float32)
    sums = lax.dot_general(onehot, h, (((1,), (0,)), ((), ())),
                           preferred_element_type=jnp.float32)
    cnt = jnp.sum(onehot, axis=1, keepdims=True)
    pooled = sums / jnp.maximum(cnt, 1.0)
    out_ref[...] = _mt(pooled, wc_ref[...]) + bc_ref[...]


@jax.jit
def kernel(x, edge_index, edge_attr, batch, W1, b1, W2, b2, W3, b3, Wc, bc):
    row = edge_index[0]
    col = edge_index[1]
    pad = EP - E
    rowm = jnp.concatenate([row, jnp.zeros((pad,), row.dtype)]
                           ).reshape(TOTCH, K)
    colm = jnp.concatenate([col, jnp.zeros((pad,), col.dtype)]
                           ).reshape(TOTCH, K)
    eap = jnp.concatenate([edge_attr, jnp.zeros((pad,), edge_attr.dtype)]
                          ).reshape(EP // K, K)
    valid = jnp.concatenate([jnp.ones((E,), jnp.float32),
                             jnp.zeros((pad,), jnp.float32)])
    validm = valid.reshape(TOTCH, K)
    valid2 = validm
    zeros_nh = jnp.zeros((N, H), jnp.float32)

    degparts = _deg_fn()(colm, validm)

    dinv_row, wedge2 = pl.pallas_call(
        _prep_body,
        out_shape=[jax.ShapeDtypeStruct((1, N), jnp.float32),
                   jax.ShapeDtypeStruct((EP // K, K), jnp.float32)],
    )(degparts, eap, valid2)
    dinvc = dinv_row.reshape(N, 1)
    wm = wedge2

    hws = pl.pallas_call(
        _mm1_body,
        out_shape=jax.ShapeDtypeStruct((N, H), jnp.float32),
    )(x, W1, dinvc)

    for (Wn, bn) in ((W2, b1), (W3, b2)):
        parts = _agg_fn()(hws, rowm, colm, wm, zeros_nh)
        hws = pl.pallas_call(
            _layer_body,
            out_shape=jax.ShapeDtypeStruct((N, H), jnp.float32),
        )(parts, hws, dinvc, Wn, bn.reshape(1, H))

    parts = _agg_fn()(hws, rowm, colm, wm, zeros_nh)
    out = pl.pallas_call(
        _final_body,
        out_shape=jax.ShapeDtypeStruct((B, C), jnp.float32),
    )(parts, hws, dinvc, b3.reshape(1, H), batch.reshape(1, N), Wc,
      bc.reshape(1, C))
    return out


# speedup vs baseline: 1.1542x; 1.0618x over previous
import functools

import jax
import jax.numpy as jnp
from jax import lax
from jax.experimental import pallas as pl
from jax.experimental.pallas import tpu as pltpu
from jax.experimental.pallas import tpu_sc as plsc

N = 10000
E = 320000
D_IN = 128
H = 64
C = 100
B = 16

NC = 2
NS = 16
NW = NC * NS
K = 128
NCHUNK = ((-(-E // (NW * K)) + 3) // 4) * 4
NCH0 = 112
NCH1 = 2 * NCHUNK - NCH0
NCHMAX = max(NCH0, NCH1)
TOTCH = NW * NCHUNK
EP = TOTCH * K
RPS = N // NS


def _mt(a, b):
    return lax.dot_general(a, b, (((1,), (1,)), ((), ())),
                           preferred_element_type=jnp.float32)


@functools.cache
def _mesh():
    return plsc.VectorSubcoreMesh(core_axis_name="c", subcore_axis_name="s",
                                  num_cores=NC, num_subcores=NS)


@functools.cache
def _deg_fn():
    @functools.partial(
        pl.kernel,
        out_type=jax.ShapeDtypeStruct((NW, N), jnp.float32),
        mesh=_mesh(),
        compiler_params=pltpu.CompilerParams(needs_layout_passes=False, use_tc_tiling_on_sc=False),
        scratch_types=[
            pltpu.VMEM((NCHUNK, K), jnp.int32),
            pltpu.VMEM((NCHUNK, K), jnp.float32),
            pltpu.VMEM((N,), jnp.float32),
        ],
    )
    def deg_kernel(colm_hbm, validm_hbm, out_hbm, colbuf, vbuf, cnt):
        c = lax.axis_index("c")
        s = lax.axis_index("s")
        w = c * NS + s
        z16 = jnp.zeros((16,), jnp.float32)

        def zb(i, carry):
            cnt[pl.ds(i * 16, 16)] = z16
            return carry
        lax.fori_loop(0, N // 16, zb, 0)

        pltpu.sync_copy(colm_hbm.at[pl.ds(w * NCHUNK, NCHUNK)], colbuf)
        pltpu.sync_copy(validm_hbm.at[pl.ds(w * NCHUNK, NCHUNK)], vbuf)

        def jb(j, carry):
            def ib(i, carry2):
                colv = colbuf[j, pl.ds(i * 16, 16)]
                vv = vbuf[j, pl.ds(i * 16, 16)]
                plsc.addupdate_scatter(cnt, [colv], vv)
                return carry2
            return lax.fori_loop(0, K // 16, ib, carry)
        lax.fori_loop(0, NCHUNK, jb, 0)

        pltpu.sync_copy(cnt, out_hbm.at[w])

    return deg_kernel


@functools.cache
def _agg_fn():
    @functools.partial(
        pl.kernel,
        out_type=jax.ShapeDtypeStruct((NC, N, H), jnp.float32),
        mesh=_mesh(),
        compiler_params=pltpu.CompilerParams(needs_layout_passes=False, use_tc_tiling_on_sc=False),
        scratch_types=[
            pltpu.VMEM((NCHMAX, K), jnp.int32),
            pltpu.VMEM((NCHMAX, K), jnp.int32),
            pltpu.VMEM((NCHMAX, K), jnp.float32),
            pltpu.VMEM((K, H), jnp.float32),
            pltpu.VMEM((K, H), jnp.float32),
            pltpu.VMEM((K, H), jnp.float32),
            pltpu.VMEM((K, H), jnp.float32),
            pltpu.VMEM_SHARED((N, H), jnp.float32),
            pltpu.SemaphoreType.DMA,
            pltpu.SemaphoreType.DMA,
            pltpu.SemaphoreType.DMA,
            pltpu.SemaphoreType.DMA,
            pltpu.SemaphoreType.DMA,
            pltpu.SemaphoreType.DMA,
            pltpu.SemaphoreType.DMA,
            pltpu.SemaphoreType.DMA,
        ],
    )
    def agg_kernel(hws_hbm, rowm_hbm, colm_hbm, wm_hbm, zeros_hbm, out_hbm,
                   rowbuf, colbuf, wbuf, g0, g1, g2, g3, acc,
                   gs0, gs1, gs2, gs3, ss0, ss1, ss2, ss3):
        c = lax.axis_index("c")
        s = lax.axis_index("s")
        w = c * NS + s
        gbufs = (g0, g1, g2, g3)
        gsems = (gs0, gs1, gs2, gs3)
        ssems = (ss0, ss1, ss2, ss3)

        pltpu.sync_copy(zeros_hbm.at[pl.ds(s * RPS, RPS)],
                        acc.at[pl.ds(s * RPS, RPS)])

        def issue_gather(slot, j):
            pltpu.async_copy(hws_hbm.at[rowbuf.at[j]], gbufs[slot],
                             gsems[slot])

        def wait_gather(slot, j):
            pltpu.make_async_copy(hws_hbm.at[rowbuf.at[j]], gbufs[slot],
                                  gsems[slot]).wait()

        def issue_scatter(slot, j):
            pltpu.async_copy(gbufs[slot], acc.at[colbuf.at[j]], ssems[slot],
                             add=True)

        def wait_scatter(slot, j):
            pltpu.make_async_copy(gbufs[slot], acc.at[colbuf.at[j]],
                                  ssems[slot]).wait()

        def scale(slot, j):
            gbuf = gbufs[slot]

            @plsc.parallel_loop(0, K // 16, unroll=4)
            def _(g):
                wv = wbuf[j, pl.ds(g * 16, 16)]
                for l in range(16):
                    sc = wv[l]
                    base = g * 16 + l
                    for q in range(H // 16):
                        gv = gbuf[base, pl.ds(q * 16, 16)]
                        gbuf[base, pl.ds(q * 16, 16)] = gv * sc

        def run_pipeline(nch, start):
            pltpu.sync_copy(rowm_hbm.at[pl.ds(start, nch)],
                            rowbuf.at[pl.ds(0, nch)])
            pltpu.sync_copy(colm_hbm.at[pl.ds(start, nch)],
                            colbuf.at[pl.ds(0, nch)])
            pltpu.sync_copy(wm_hbm.at[pl.ds(start, nch)],
                            wbuf.at[pl.ds(0, nch)])
            issue_gather(0, 0)
            issue_gather(1, 1)

            def body(i, carry):
                for b in range(4):
                    j = 4 * i + b
                    s2 = (b + 2) % 4
                    if b < 2:
                        @pl.when(i > 0)
                        def _():
                            wait_scatter(s2, j - 2)
                        issue_gather(s2, j + 2)
                    else:
                        wait_scatter(s2, j - 2)

                        @pl.when(i < nch // 4 - 1)
                        def _():
                            issue_gather(s2, j + 2)
                    wait_gather(b, j)
                    scale(b, j)
                    issue_scatter(b, j)
                return carry
            lax.fori_loop(0, nch // 4, body, 0)
            wait_scatter(2, nch - 2)
            wait_scatter(3, nch - 1)

        pair_base = s * (NCH0 + NCH1)

        @pl.when(c == 0)
        def _():
            run_pipeline(NCH0, pair_base)

        @pl.when(c == 1)
        def _():
            run_pipeline(NCH1, pair_base + NCH0)

        plsc.subcore_barrier()
        pltpu.sync_copy(acc.at[pl.ds(s * RPS, RPS)],
                        out_hbm.at[c, pl.ds(s * RPS, RPS)])

    return agg_kernel


def _prep_body(degparts_ref, eap_ref, valid_ref, dinv_ref, wedge_ref):
    deg = jnp.sum(degparts_ref[...], axis=0, keepdims=True) + 1.0
    dinv_ref[...] = lax.rsqrt(deg)
    wedge_ref[...] = jnp.exp(-eap_ref[...]) * valid_ref[...]


def _mm1_body(x_ref, w1_ref, dinvc_ref, out_ref):
    out_ref[...] = _mt(x_ref[...], w1_ref[...]) * dinvc_ref[...]


def _layer_body(p_ref, hws_ref, dinvc_ref, w_ref, b_ref, out_ref):
    t = p_ref[0] + p_ref[1] + hws_ref[...]
    h = jnp.maximum(dinvc_ref[...] * t + b_ref[...], 0.0)
    out_ref[...] = _mt(h, w_ref[...]) * dinvc_ref[...]


def _final_body(p_ref, hws_ref, dinvc_ref, b_ref, batch_ref, wc_ref, bc_ref,
                out_ref):
    t = p_ref[0] + p_ref[1] + hws_ref[...]
    h = jnp.maximum(dinvc_ref[...] * t + b_ref[...], 0.0)
    ids = lax.broadcasted_iota(jnp.int32, (B, N), 0)
    onehot = (batch_ref[...] == ids).astype(jnp.float32)
    sums = lax.dot_general(onehot, h, (((1,), (0,)), ((), ())),
                           preferred_element_type=jnp.float32)
    cnt = jnp.sum(onehot, axis=1, keepdims=True)
    pooled = sums / jnp.maximum(cnt, 1.0)
    out_ref[...] = _mt(pooled, wc_ref[...]) + bc_ref[...]


@jax.jit
def kernel(x, edge_index, edge_attr, batch, W1, b1, W2, b2, W3, b3, Wc, bc):
    row = edge_index[0]
    col = edge_index[1]
    pad = EP - E
    rowm = jnp.concatenate([row, jnp.zeros((pad,), row.dtype)]
                           ).reshape(TOTCH, K)
    colm = jnp.concatenate([col, jnp.zeros((pad,), col.dtype)]
                           ).reshape(TOTCH, K)
    eap = jnp.concatenate([edge_attr, jnp.zeros((pad,), edge_attr.dtype)]
                          ).reshape(EP // K, K)
    valid = jnp.concatenate([jnp.ones((E,), jnp.float32),
                             jnp.zeros((pad,), jnp.float32)])
    validm = valid.reshape(TOTCH, K)
    valid2 = validm
    zeros_nh = jnp.zeros((N, H), jnp.float32)

    degparts = _deg_fn()(colm, validm)

    dinv_row, wedge2 = pl.pallas_call(
        _prep_body,
        out_shape=[jax.ShapeDtypeStruct((1, N), jnp.float32),
                   jax.ShapeDtypeStruct((EP // K, K), jnp.float32)],
    )(degparts, eap, valid2)
    dinvc = dinv_row.reshape(N, 1)
    wm = wedge2

    hws = pl.pallas_call(
        _mm1_body,
        out_shape=jax.ShapeDtypeStruct((N, H), jnp.float32),
    )(x, W1, dinvc)

    for (Wn, bn) in ((W2, b1), (W3, b2)):
        parts = _agg_fn()(hws, rowm, colm, wm, zeros_nh)
        hws = pl.pallas_call(
            _layer_body,
            out_shape=jax.ShapeDtypeStruct((N, H), jnp.float32),
        )(parts, hws, dinvc, Wn, bn.reshape(1, H))

    parts = _agg_fn()(hws, rowm, colm, wm, zeros_nh)
    out = pl.pallas_call(
        _final_body,
        out_shape=jax.ShapeDtypeStruct((B, C), jnp.float32),
    )(parts, hws, dinvc, b3.reshape(1, H), batch.reshape(1, N), Wc,
      bc.reshape(1, C))
    return out


# speedup vs baseline: 1.1962x; 1.0364x over previous
import functools

import jax
import jax.numpy as jnp
from jax import lax
from jax.experimental import pallas as pl
from jax.experimental.pallas import tpu as pltpu
from jax.experimental.pallas import tpu_sc as plsc

N = 10000
E = 320000
D_IN = 128
H = 64
C = 100
B = 16

NC = 2
NS = 16
NW = NC * NS
K = 128
NCHUNK = ((-(-E // (NW * K)) + 3) // 4) * 4
NCH0 = 128
NCH1 = 2 * NCHUNK - NCH0
NCHMAX = max(NCH0, NCH1)
TOTCH = NW * NCHUNK
EP = TOTCH * K
RPS = N // NS


def _mt(a, b):
    return lax.dot_general(a, b, (((1,), (1,)), ((), ())),
                           preferred_element_type=jnp.float32)


@functools.cache
def _mesh():
    return plsc.VectorSubcoreMesh(core_axis_name="c", subcore_axis_name="s",
                                  num_cores=NC, num_subcores=NS)


@functools.cache
def _deg_fn():
    @functools.partial(
        pl.kernel,
        out_type=jax.ShapeDtypeStruct((NW, N), jnp.float32),
        mesh=_mesh(),
        compiler_params=pltpu.CompilerParams(needs_layout_passes=False, use_tc_tiling_on_sc=False),
        scratch_types=[
            pltpu.VMEM((NCHUNK, K), jnp.int32),
            pltpu.VMEM((NCHUNK, K), jnp.float32),
            pltpu.VMEM((N,), jnp.float32),
        ],
    )
    def deg_kernel(colm_hbm, validm_hbm, out_hbm, colbuf, vbuf, cnt):
        c = lax.axis_index("c")
        s = lax.axis_index("s")
        w = c * NS + s
        z16 = jnp.zeros((16,), jnp.float32)

        def zb(i, carry):
            cnt[pl.ds(i * 16, 16)] = z16
            return carry
        lax.fori_loop(0, N // 16, zb, 0)

        pltpu.sync_copy(colm_hbm.at[pl.ds(w * NCHUNK, NCHUNK)], colbuf)
        pltpu.sync_copy(validm_hbm.at[pl.ds(w * NCHUNK, NCHUNK)], vbuf)

        def jb(j, carry):
            def ib(i, carry2):
                colv = colbuf[j, pl.ds(i * 16, 16)]
                vv = vbuf[j, pl.ds(i * 16, 16)]
                plsc.addupdate_scatter(cnt, [colv], vv)
                return carry2
            return lax.fori_loop(0, K // 16, ib, carry)
        lax.fori_loop(0, NCHUNK, jb, 0)

        pltpu.sync_copy(cnt, out_hbm.at[w])

    return deg_kernel


@functools.cache
def _agg_fn():
    @functools.partial(
        pl.kernel,
        out_type=jax.ShapeDtypeStruct((NC, N, H), jnp.float32),
        mesh=_mesh(),
        compiler_params=pltpu.CompilerParams(needs_layout_passes=False, use_tc_tiling_on_sc=False),
        scratch_types=[
            pltpu.VMEM((NCHMAX, K), jnp.int32),
            pltpu.VMEM((NCHMAX, K), jnp.int32),
            pltpu.VMEM((NCHMAX, K), jnp.float32),
            pltpu.VMEM((K, H), jnp.float32),
            pltpu.VMEM((K, H), jnp.float32),
            pltpu.VMEM((K, H), jnp.float32),
            pltpu.VMEM((K, H), jnp.float32),
            pltpu.VMEM_SHARED((N, H), jnp.float32),
            pltpu.SemaphoreType.DMA,
            pltpu.SemaphoreType.DMA,
            pltpu.SemaphoreType.DMA,
            pltpu.SemaphoreType.DMA,
            pltpu.SemaphoreType.DMA,
            pltpu.SemaphoreType.DMA,
            pltpu.SemaphoreType.DMA,
            pltpu.SemaphoreType.DMA,
        ],
    )
    def agg_kernel(hws_hbm, rowm_hbm, colm_hbm, wm_hbm, zeros_hbm, out_hbm,
                   rowbuf, colbuf, wbuf, g0, g1, g2, g3, acc,
                   gs0, gs1, gs2, gs3, ss0, ss1, ss2, ss3):
        c = lax.axis_index("c")
        s = lax.axis_index("s")
        w = c * NS + s
        gbufs = (g0, g1, g2, g3)
        gsems = (gs0, gs1, gs2, gs3)
        ssems = (ss0, ss1, ss2, ss3)

        pltpu.sync_copy(zeros_hbm.at[pl.ds(s * RPS, RPS)],
                        acc.at[pl.ds(s * RPS, RPS)])

        def issue_gather(slot, j):
            pltpu.async_copy(hws_hbm.at[rowbuf.at[j]], gbufs[slot],
                             gsems[slot])

        def wait_gather(slot, j):
            pltpu.make_async_copy(hws_hbm.at[rowbuf.at[j]], gbufs[slot],
                                  gsems[slot]).wait()

        def issue_scatter(slot, j):
            pltpu.async_copy(gbufs[slot], acc.at[colbuf.at[j]], ssems[slot],
                             add=True)

        def wait_scatter(slot, j):
            pltpu.make_async_copy(gbufs[slot], acc.at[colbuf.at[j]],
                                  ssems[slot]).wait()

        def scale(slot, j):
            gbuf = gbufs[slot]

            @plsc.parallel_loop(0, K // 16, unroll=4)
            def _(g):
                wv = wbuf[j, pl.ds(g * 16, 16)]
                for l in range(16):
                    sc = wv[l]
                    base = g * 16 + l
                    for q in range(H // 16):
                        gv = gbuf[base, pl.ds(q * 16, 16)]
                        gbuf[base, pl.ds(q * 16, 16)] = gv * sc

        def run_pipeline(nch, start):
            pltpu.sync_copy(rowm_hbm.at[pl.ds(start, nch)],
                            rowbuf.at[pl.ds(0, nch)])
            pltpu.sync_copy(colm_hbm.at[pl.ds(start, nch)],
                            colbuf.at[pl.ds(0, nch)])
            pltpu.sync_copy(wm_hbm.at[pl.ds(start, nch)],
                            wbuf.at[pl.ds(0, nch)])
            issue_gather(0, 0)
            issue_gather(1, 1)

            def body(i, carry):
                for b in range(4):
                    j = 4 * i + b
                    s2 = (b + 2) % 4
                    if b < 2:
                        @pl.when(i > 0)
                        def _():
                            wait_scatter(s2, j - 2)
                        issue_gather(s2, j + 2)
                    else:
                        wait_scatter(s2, j - 2)

                        @pl.when(i < nch // 4 - 1)
                        def _():
                            issue_gather(s2, j + 2)
                    wait_gather(b, j)
                    scale(b, j)
                    issue_scatter(b, j)
                return carry
            lax.fori_loop(0, nch // 4, body, 0)
            wait_scatter(2, nch - 2)
            wait_scatter(3, nch - 1)

        pair_base = s * (NCH0 + NCH1)

        @pl.when(c == 0)
        def _():
            run_pipeline(NCH0, pair_base)

        @pl.when(c == 1)
        def _():
            run_pipeline(NCH1, pair_base + NCH0)

        plsc.subcore_barrier()
        pltpu.sync_copy(acc.at[pl.ds(s * RPS, RPS)],
                        out_hbm.at[c, pl.ds(s * RPS, RPS)])

    return agg_kernel


def _prep_body(degparts_ref, eap_ref, valid_ref, dinv_ref, wedge_ref):
    deg = jnp.sum(degparts_ref[...], axis=0, keepdims=True) + 1.0
    dinv_ref[...] = lax.rsqrt(deg)
    wedge_ref[...] = jnp.exp(-eap_ref[...]) * valid_ref[...]


def _mm1_body(x_ref, w1_ref, dinvc_ref, out_ref):
    out_ref[...] = _mt(x_ref[...], w1_ref[...]) * dinvc_ref[...]


def _layer_body(p_ref, hws_ref, dinvc_ref, w_ref, b_ref, out_ref):
    t = p_ref[0] + p_ref[1] + hws_ref[...]
    h = jnp.maximum(dinvc_ref[...] * t + b_ref[...], 0.0)
    out_ref[...] = _mt(h, w_ref[...]) * dinvc_ref[...]


def _final_body(p_ref, hws_ref, dinvc_ref, b_ref, batch_ref, wc_ref, bc_ref,
                out_ref):
    t = p_ref[0] + p_ref[1] + hws_ref[...]
    h = jnp.maximum(dinvc_ref[...] * t + b_ref[...], 0.0)
    ids = lax.broadcasted_iota(jnp.int32, (B, N), 0)
    onehot = (batch_ref[...] == ids).astype(jnp.float32)
    sums = lax.dot_general(onehot, h, (((1,), (0,)), ((), ())),
                           preferred_element_type=jnp.float32)
    cnt = jnp.sum(onehot, axis=1, keepdims=True)
    pooled = sums / jnp.maximum(cnt, 1.0)
    out_ref[...] = _mt(pooled, wc_ref[...]) + bc_ref[...]


@jax.jit
def kernel(x, edge_index, edge_attr, batch, W1, b1, W2, b2, W3, b3, Wc, bc):
    row = edge_index[0]
    col = edge_index[1]
    pad = EP - E
    rowm = jnp.concatenate([row, jnp.zeros((pad,), row.dtype)]
                           ).reshape(TOTCH, K)
    colm = jnp.concatenate([col, jnp.zeros((pad,), col.dtype)]
                           ).reshape(TOTCH, K)
    eap = jnp.concatenate([edge_attr, jnp.zeros((pad,), edge_attr.dtype)]
                          ).reshape(EP // K, K)
    valid = jnp.concatenate([jnp.ones((E,), jnp.float32),
                             jnp.zeros((pad,), jnp.float32)])
    validm = valid.reshape(TOTCH, K)
    valid2 = validm
    zeros_nh = jnp.zeros((N, H), jnp.float32)

    degparts = _deg_fn()(colm, validm)

    dinv_row, wedge2 = pl.pallas_call(
        _prep_body,
        out_shape=[jax.ShapeDtypeStruct((1, N), jnp.float32),
                   jax.ShapeDtypeStruct((EP // K, K), jnp.float32)],
    )(degparts, eap, valid2)
    dinvc = dinv_row.reshape(N, 1)
    wm = wedge2

    hws = pl.pallas_call(
        _mm1_body,
        out_shape=jax.ShapeDtypeStruct((N, H), jnp.float32),
    )(x, W1, dinvc)

    for (Wn, bn) in ((W2, b1), (W3, b2)):
        parts = _agg_fn()(hws, rowm, colm, wm, zeros_nh)
        hws = pl.pallas_call(
            _layer_body,
            out_shape=jax.ShapeDtypeStruct((N, H), jnp.float32),
        )(parts, hws, dinvc, Wn, bn.reshape(1, H))

    parts = _agg_fn()(hws, rowm, colm, wm, zeros_nh)
    out = pl.pallas_call(
        _final_body,
        out_shape=jax.ShapeDtypeStruct((B, C), jnp.float32),
    )(parts, hws, dinvc, b3.reshape(1, H), batch.reshape(1, N), Wc,
      bc.reshape(1, C))
    return out
